# Initial kernel scaffold; baseline (speedup 1.0000x reference)
import jax
import jax.numpy as jnp
from jax.experimental import pallas as pl


def kernel(encoded, Wa, ba, Wb, bb):
    raise NotImplementedError("write your pallas kernel here")


# speedup vs baseline: 1.0419x; 1.0419x over previous
import functools

import jax
import jax.numpy as jnp
from jax import lax
from jax.experimental import pallas as pl
from jax.experimental.pallas import tpu as pltpu

_NEG_INF = float("-inf")


def _wmax9(zp, axis):
    n = zp.shape[axis] - 8

    def sl(z, start, length):
        idx = [slice(None)] * z.ndim
        idx[axis] = slice(start, start + length)
        return z[tuple(idx)]

    a = jnp.maximum(sl(zp, 0, n + 7), sl(zp, 1, n + 7))
    b = jnp.maximum(sl(a, 0, n + 5), sl(a, 2, n + 5))
    c = jnp.maximum(sl(b, 0, n + 1), sl(b, 4, n + 1))
    return jnp.maximum(sl(c, 0, n), sl(zp, 8, n))


def _maxpool9(x):
    pad_r = jnp.full((4, x.shape[1]), _NEG_INF, x.dtype)
    xp = jnp.concatenate([pad_r, x, pad_r], axis=0)
    x1 = _wmax9(xp, 0)
    pad_c = jnp.full((x.shape[0], 4), _NEG_INF, x.dtype)
    xp2 = jnp.concatenate([pad_c, x1, pad_c], axis=1)
    return _wmax9(xp2, 1)


def _a1_body(x_ref, w1_ref, ba_ref, w2_ref, bb_ref, out_ref):
    x = x_ref[...]
    w_lane = lax.broadcasted_iota(jnp.int32, (1, 4096), 1) % 64
    zero_col = jnp.zeros((128, 72), jnp.float32)
    xp = jnp.concatenate([zero_col, x, zero_col], axis=1)
    taps = []
    for dy in range(3):
        for dx in range(3):
            s = (dy - 1) * 64 + (dx - 1)
            sh = xp[:, 72 + s:72 + s + 4096]
            wv = w_lane + (dx - 1)
            m = ((wv >= 0) & (wv < 64)).astype(jnp.float32)
            taps.append(sh * m)
    x9 = jnp.stack(taps, axis=0)
    x9 = jnp.transpose(x9, (1, 0, 2))
    x9 = jnp.reshape(x9, (1152, 4096))
    y1 = jnp.dot(w1_ref[...], x9, preferred_element_type=jnp.float32)
    y1 = jnp.maximum(y1 + ba_ref[...], 0.0)
    s = jnp.dot(w2_ref[...], y1, preferred_element_type=jnp.float32)
    s = s + bb_ref[...]
    m = jnp.max(s, axis=0, keepdims=True)
    e = jnp.exp(s - m)
    acc = e[0:1]
    for c in range(1, 65):
        acc = acc + e[c:c + 1]
    out_ref[...] = (e / acc)[:64]


def _a2_body(s_ref, masked_ref, meta_ref):
    s = s_ref[...]
    zeros = jnp.zeros_like(s)
    max_mask = s == _maxpool9(s)
    for _ in range(2):
        supp_mask = _maxpool9(max_mask.astype(jnp.float32)) > 0
        supp_scores = jnp.where(supp_mask, zeros, s)
        new_max_mask = supp_scores == _maxpool9(supp_scores)
        max_mask = max_mask | (new_max_mask & (~supp_mask))
    nms = jnp.where(max_mask, s, zeros)
    rows = lax.broadcasted_iota(jnp.int32, (512, 512), 0)
    cols = lax.broadcasted_iota(jnp.int32, (512, 512), 1)
    border = ((rows >= 4) & (rows < 508) & (cols >= 4) & (cols < 508))
    valid = (nms > 0.005) & border
    masked = jnp.where(valid, nms, 0.0)
    masked_ref[...] = masked

    bits = lax.bitcast_convert_type(masked, jnp.int32)
    p_cnt = jnp.sum((bits >= 1).astype(jnp.int32))

    def bs_body(i, t):
        cand = t | (jnp.int32(1) << (jnp.int32(29) - i))
        cnt = jnp.sum((bits >= cand).astype(jnp.int32))
        return jnp.where(cnt >= 1024, cand, t)

    t_star = lax.fori_loop(0, 30, bs_body, jnp.int32(0))
    lane = lax.broadcasted_iota(jnp.int32, (1, 128), 1)
    meta = jnp.where(lane == 0, t_star, jnp.where(lane == 1, p_cnt, 0))
    meta_ref[...] = meta


def _scores_pipeline(encoded, Wa, ba, Wb, bb, interpret=False):
    x = encoded.reshape(128, 4096)
    w1 = Wa.reshape(256, 1152)
    w2 = Wb.reshape(65, 256)
    a1 = pl.pallas_call(
        _a1_body,
        out_shape=jax.ShapeDtypeStruct((64, 4096), jnp.float32),
        interpret=interpret,
    )
    s_small = a1(x, w1, ba.reshape(256, 1), w2, bb.reshape(65, 1))
    scores = (s_small.reshape(8, 8, 64, 64)
              .transpose(2, 0, 3, 1).reshape(512, 512))
    a2 = pl.pallas_call(
        _a2_body,
        out_shape=(jax.ShapeDtypeStruct((512, 512), jnp.float32),
                   jax.ShapeDtypeStruct((1, 128), jnp.int32)),
        interpret=interpret,
    )
    return a2(scores)


def kernel(encoded, Wa, ba, Wb, bb):
    masked, meta = _scores_pipeline(encoded, Wa, ba, Wb, bb)
    flat = masked.reshape(-1)
    top_scores, top_idx = lax.top_k(flat, 1024)
    ky = top_idx // 512
    kx = top_idx % 512
    keypoints = jnp.stack([kx, ky], axis=1).astype(jnp.float32)
    return (keypoints, top_scores)
